# baseline (speedup 1.0000x reference)
import jax
import jax.numpy as jnp
import numpy as np
from jax.experimental import pallas as pl

_T, _B, _D = 2048, 32, 768
_H, _DK = 12, 64
_TILE_T = 32
_R = _TILE_T * _B
_AB = 256


def _mha_body(x_ref, wq_ref, wk_ref, wv_ref, wo_ref, bias_ref, me_ref,
              ones_ref, out_ref):
    x = x_ref[...]
    q = jnp.dot(x, wq_ref[...],
                preferred_element_type=jnp.float32).astype(jnp.bfloat16)
    k = jnp.dot(x, wk_ref[...],
                preferred_element_type=jnp.float32).astype(jnp.bfloat16)
    v = jnp.dot(x, wv_ref[...],
                preferred_element_type=jnp.float32).astype(jnp.bfloat16)
    bias = bias_ref[...]
    me = me_ref[...]
    onesc = ones_ref[...]
    sls = [slice(h * _DK, (h + 1) * _DK) for h in range(_H)]
    for b0 in range(0, _R, _AB):
        rb = slice(b0, b0 + _AB)
        ss = [jax.lax.dot_general(q[rb, sl], k[rb, sl],
                                  (((1,), (1,)), ((), ())),
                                  preferred_element_type=jnp.float32) + bias
              for sl in sls]
        ms = [jnp.max(s, axis=-1, keepdims=True) for s in ss]
        es = [jnp.exp(s - m).astype(jnp.bfloat16) for s, m in zip(ss, ms)]
        vps = []
        for p in range(_H // 2):
            vpair = v[rb, 2 * p * _DK:(2 * p + 2) * _DK]
            vp_top = vpair * me
            vp_bot = vpair - vp_top
            vps.append(jnp.concatenate(
                [jnp.concatenate([vp_top, vp_bot], axis=0), onesc], axis=1))
        ous = [jnp.dot(jnp.concatenate([es[2 * p], es[2 * p + 1]], axis=1),
                       vps[p], preferred_element_type=jnp.float32)
               for p in range(_H // 2)]
        ops = [(ou[:, :2 * _DK] * (1.0 / ou[:, 2 * _DK:]))
               .astype(jnp.bfloat16) for ou in ous]
        o = jnp.concatenate(ops, axis=-1)
        out_ref[rb, :] = jnp.dot(o, wo_ref[...],
                                 preferred_element_type=jnp.float32)


def _metadata(mask):
    conn = (mask[:, 0] == 0.0).any(axis=0)
    conn = conn | jnp.eye(_B, dtype=bool)
    n_iter = max(1, int(np.ceil(np.log2(max(_B, 2)))))
    for _ in range(n_iter):
        conn = conn | jnp.any(conn[:, :, None] & conn[None, :, :], axis=1)
    gids = jnp.min(jnp.where(conn, jnp.arange(_B), _B), axis=1)
    allow0 = (gids[:, None] == gids[None, :]) | jnp.eye(_B, dtype=bool)
    rb = jnp.arange(_AB) % _B
    rt = jnp.arange(_AB) // _B
    same_t = rt[:, None] == rt[None, :]
    allow_big = allow0[rb[:, None], rb[None, :]] & same_t
    bias = jnp.where(allow_big, 0.0, -1e30).astype(jnp.float32)
    lane = jnp.arange(2 * _DK)
    me = (lane[None, :] < _DK).astype(jnp.bfloat16) * jnp.ones((_AB, 1),
                                                               jnp.bfloat16)
    row = jnp.arange(2 * _AB)
    onesc = (((row[:, None] < _AB) & (lane[None, :] < _DK))
             | ((row[:, None] >= _AB) & (lane[None, :] >= _DK))
             ).astype(jnp.bfloat16)
    return bias, me, onesc


def kernel(hidden_states, mask, Wq, Wk, Wv, Wo):
    bias, me, onesc = _metadata(mask)
    x2 = hidden_states.reshape(_T * _B, _D).astype(jnp.bfloat16)
    scale = np.float32(1.0 / np.sqrt(_DK))
    wq = (Wq * scale).T.astype(jnp.bfloat16)
    wk, wv, wo = (w.T.astype(jnp.bfloat16) for w in (Wk, Wv, Wo))
    wspec = pl.BlockSpec((_D, _D), lambda i: (0, 0))
    out = pl.pallas_call(
        _mha_body,
        grid=(_T // _TILE_T,),
        in_specs=[
            pl.BlockSpec((_R, _D), lambda i: (i, 0)),
            wspec, wspec, wspec, wspec,
            pl.BlockSpec((_AB, _AB), lambda i: (0, 0)),
            pl.BlockSpec((_AB, 2 * _DK), lambda i: (0, 0)),
            pl.BlockSpec((2 * _AB, 2 * _DK), lambda i: (0, 0)),
        ],
        out_specs=pl.BlockSpec((_R, _D), lambda i: (i, 0)),
        out_shape=jax.ShapeDtypeStruct((_T * _B, _D), jnp.float32),
    )(x2, wq, wk, wv, wo, bias, me, onesc)
    return out.reshape(_T, _B, _D)


# speedup vs baseline: 1.0704x; 1.0704x over previous
import jax
import jax.numpy as jnp
import numpy as np
from jax.experimental import pallas as pl
from jax.experimental.pallas import tpu as pltpu

_T, _B, _D = 2048, 32, 768
_H, _DK = 12, 64
_TILE_T = 32
_R = _TILE_T * _B
_AB = 256


def _mha_body(x_ref, wq_ref, wk_ref, wv_ref, wo_ref, bias_ref, me_ref,
              ones_ref, out_ref):
    x = x_ref[...].astype(jnp.bfloat16)
    q = jnp.dot(x, wq_ref[...],
                preferred_element_type=jnp.float32).astype(jnp.bfloat16)
    k = jnp.dot(x, wk_ref[...],
                preferred_element_type=jnp.float32).astype(jnp.bfloat16)
    v = jnp.dot(x, wv_ref[...],
                preferred_element_type=jnp.float32).astype(jnp.bfloat16)
    bias = bias_ref[...]
    me = me_ref[...]
    onesc = ones_ref[...]
    sls = [slice(h * _DK, (h + 1) * _DK) for h in range(_H)]
    for b0 in range(0, _R, _AB):
        rb = slice(b0, b0 + _AB)
        ss = [jax.lax.dot_general(q[rb, sl], k[rb, sl],
                                  (((1,), (1,)), ((), ())),
                                  preferred_element_type=jnp.float32) + bias
              for sl in sls]
        ms = [jnp.max(s, axis=-1, keepdims=True) for s in ss]
        es = [jnp.exp(s - m).astype(jnp.bfloat16) for s, m in zip(ss, ms)]
        vps = []
        for p in range(_H // 2):
            vpair = v[rb, 2 * p * _DK:(2 * p + 2) * _DK]
            vp_top = vpair * me
            vp_bot = vpair - vp_top
            vps.append(jnp.concatenate(
                [jnp.concatenate([vp_top, vp_bot], axis=0), onesc], axis=1))
        ous = [jnp.dot(jnp.concatenate([es[2 * p], es[2 * p + 1]], axis=1),
                       vps[p], preferred_element_type=jnp.float32)
               for p in range(_H // 2)]
        ops = [(ou[:, :2 * _DK] * (1.0 / ou[:, 2 * _DK:]))
               .astype(jnp.bfloat16) for ou in ous]
        o = jnp.concatenate(ops, axis=-1)
        out_ref[rb, :] = jnp.dot(o, wo_ref[...],
                                 preferred_element_type=jnp.float32)


def _metadata(mask):
    conn = (mask[:, 0] == 0.0).any(axis=0)
    conn = conn | jnp.eye(_B, dtype=bool)
    n_iter = max(1, int(np.ceil(np.log2(max(_B, 2)))))
    for _ in range(n_iter):
        conn = conn | jnp.any(conn[:, :, None] & conn[None, :, :], axis=1)
    gids = jnp.min(jnp.where(conn, jnp.arange(_B), _B), axis=1)
    allow0 = (gids[:, None] == gids[None, :]) | jnp.eye(_B, dtype=bool)
    rb = jnp.arange(_AB) % _B
    rt = jnp.arange(_AB) // _B
    same_t = rt[:, None] == rt[None, :]
    allow_big = allow0[rb[:, None], rb[None, :]] & same_t
    bias = jnp.where(allow_big, 0.0, -1e30).astype(jnp.float32)
    lane = jnp.arange(2 * _DK)
    me = (lane[None, :] < _DK).astype(jnp.bfloat16) * jnp.ones((_AB, 1),
                                                               jnp.bfloat16)
    row = jnp.arange(2 * _AB)
    onesc = (((row[:, None] < _AB) & (lane[None, :] < _DK))
             | ((row[:, None] >= _AB) & (lane[None, :] >= _DK))
             ).astype(jnp.bfloat16)
    return bias, me, onesc


def kernel(hidden_states, mask, Wq, Wk, Wv, Wo):
    bias, me, onesc = _metadata(mask)
    x2 = hidden_states.reshape(_T * _B, _D)
    scale = np.float32(1.0 / np.sqrt(_DK))
    wq = (Wq * scale).T.astype(jnp.bfloat16)
    wk, wv, wo = (w.T.astype(jnp.bfloat16) for w in (Wk, Wv, Wo))
    wspec = pl.BlockSpec((_D, _D), lambda i: (0, 0))
    out = pl.pallas_call(
        _mha_body,
        grid=(_T // _TILE_T,),
        compiler_params=pltpu.CompilerParams(
            dimension_semantics=(pltpu.PARALLEL,)),
        in_specs=[
            pl.BlockSpec((_R, _D), lambda i: (i, 0)),
            wspec, wspec, wspec, wspec,
            pl.BlockSpec((_AB, _AB), lambda i: (0, 0)),
            pl.BlockSpec((_AB, 2 * _DK), lambda i: (0, 0)),
            pl.BlockSpec((2 * _AB, 2 * _DK), lambda i: (0, 0)),
        ],
        out_specs=pl.BlockSpec((_R, _D), lambda i: (i, 0)),
        out_shape=jax.ShapeDtypeStruct((_T * _B, _D), jnp.float32),
    )(x2, wq, wk, wv, wo, bias, me, onesc)
    return out.reshape(_T, _B, _D)
